# Initial kernel scaffold; baseline (speedup 1.0000x reference)
import jax
import jax.numpy as jnp
from jax.experimental import pallas as pl


def kernel(x, router_logits, w1, w3, w2):
    raise NotImplementedError("write your pallas kernel here")


# speedup vs baseline: 1.1689x; 1.1689x over previous
import functools
import jax
import jax.numpy as jnp
from jax.experimental import pallas as pl
from jax.experimental.pallas import tpu as pltpu

_NUM_EXPERTS = 8
_TOP_K = 2
_HIDDEN = 1024
_INTER = 2048
_TOKENS = 2048

_BT = 512
_BI = 512


def _moe_body(logits_ref, x_ref, w1_ref, w3_ref, w2_ref, out_ref):
    e = pl.program_id(1)
    i = pl.program_id(2)

    @pl.when(jnp.logical_and(e == 0, i == 0))
    def _():
        out_ref[...] = jnp.zeros_like(out_ref)

    l = logits_ref[...]
    iota8 = jax.lax.broadcasted_iota(jnp.int32, l.shape, 1)
    m1 = jnp.max(l, axis=1, keepdims=True)
    idx0 = jnp.min(jnp.where(l >= m1, iota8, _NUM_EXPERTS), axis=1, keepdims=True)
    l2 = jnp.where(iota8 == idx0, -jnp.inf, l)
    m2 = jnp.max(l2, axis=1, keepdims=True)
    idx1 = jnp.min(jnp.where(l2 >= m2, iota8, _NUM_EXPERTS), axis=1, keepdims=True)
    p1 = jnp.exp(m2 - m1)
    g0 = 1.0 / (1.0 + p1)
    g1 = p1 * g0
    cw_e = jnp.where(idx0 == e, g0, jnp.where(idx1 == e, g1, 0.0))

    x = x_ref[...]
    w1 = w1_ref[0]
    w3 = w3_ref[0]
    w2 = w2_ref[0]

    dn = (((1,), (1,)), ((), ()))
    gate = jax.lax.dot_general(x, w1, dn, preferred_element_type=jnp.float32)
    up = jax.lax.dot_general(x, w3, dn, preferred_element_type=jnp.float32)
    h = (gate * jax.lax.logistic(gate)) * up
    y = jax.lax.dot_general(h, w2, dn, preferred_element_type=jnp.float32)
    out_ref[...] += cw_e * y


def kernel(x, router_logits, w1, w3, w2):
    grid = (_TOKENS // _BT, _NUM_EXPERTS, _INTER // _BI)
    return pl.pallas_call(
        _moe_body,
        grid=grid,
        in_specs=[
            pl.BlockSpec((_BT, _NUM_EXPERTS), lambda t, e, i: (t, 0)),
            pl.BlockSpec((_BT, _HIDDEN), lambda t, e, i: (t, 0)),
            pl.BlockSpec((1, _BI, _HIDDEN), lambda t, e, i: (e, i, 0)),
            pl.BlockSpec((1, _BI, _HIDDEN), lambda t, e, i: (e, i, 0)),
            pl.BlockSpec((1, _HIDDEN, _BI), lambda t, e, i: (e, 0, i)),
        ],
        out_specs=pl.BlockSpec((_BT, _HIDDEN), lambda t, e, i: (t, 0)),
        out_shape=jax.ShapeDtypeStruct((_TOKENS, _HIDDEN), jnp.float32),
        compiler_params=pltpu.CompilerParams(
            dimension_semantics=("parallel", "arbitrary", "arbitrary"),
        ),
    )(router_logits, x, w1, w3, w2)


# speedup vs baseline: 1.1816x; 1.0109x over previous
import functools
import jax
import jax.numpy as jnp
from jax.experimental import pallas as pl
from jax.experimental.pallas import tpu as pltpu

_NUM_EXPERTS = 8
_TOP_K = 2
_HIDDEN = 1024
_INTER = 2048
_TOKENS = 2048

_BT = 512
_BI = 512


def _moe_body(logits_ref, x_ref, w1_ref, w3_ref, w2_ref, out_ref):
    e = pl.program_id(1)
    i = pl.program_id(2)

    @pl.when(jnp.logical_and(e == 0, i == 0))
    def _():
        out_ref[...] = jnp.zeros_like(out_ref)

    l = logits_ref[...]
    iota8 = jax.lax.broadcasted_iota(jnp.int32, l.shape, 1)
    m1 = jnp.max(l, axis=1, keepdims=True)
    idx0 = jnp.min(jnp.where(l >= m1, iota8, _NUM_EXPERTS), axis=1, keepdims=True)
    l2 = jnp.where(iota8 == idx0, -jnp.inf, l)
    m2 = jnp.max(l2, axis=1, keepdims=True)
    idx1 = jnp.min(jnp.where(l2 >= m2, iota8, _NUM_EXPERTS), axis=1, keepdims=True)
    p1 = jnp.exp(m2 - m1)
    g0 = 1.0 / (1.0 + p1)
    g1 = p1 * g0
    cw_e = jnp.where(idx0 == e, g0, jnp.where(idx1 == e, g1, 0.0))

    x = x_ref[...]
    w1 = w1_ref[0]
    w3 = w3_ref[0]
    w2 = w2_ref[0]

    dn = (((1,), (1,)), ((), ()))
    xb = x.astype(jnp.bfloat16)
    gate = jax.lax.dot_general(xb, w1.astype(jnp.bfloat16), dn,
                               preferred_element_type=jnp.float32)
    up = jax.lax.dot_general(xb, w3.astype(jnp.bfloat16), dn,
                             preferred_element_type=jnp.float32)
    h = (gate * jax.lax.logistic(gate)) * up
    y = jax.lax.dot_general(h.astype(jnp.bfloat16), w2.astype(jnp.bfloat16), dn,
                            preferred_element_type=jnp.float32)
    out_ref[...] += cw_e * y


def kernel(x, router_logits, w1, w3, w2):
    grid = (_TOKENS // _BT, _NUM_EXPERTS, _INTER // _BI)
    return pl.pallas_call(
        _moe_body,
        grid=grid,
        in_specs=[
            pl.BlockSpec((_BT, _NUM_EXPERTS), lambda t, e, i: (t, 0)),
            pl.BlockSpec((_BT, _HIDDEN), lambda t, e, i: (t, 0)),
            pl.BlockSpec((1, _BI, _HIDDEN), lambda t, e, i: (e, i, 0)),
            pl.BlockSpec((1, _BI, _HIDDEN), lambda t, e, i: (e, i, 0)),
            pl.BlockSpec((1, _HIDDEN, _BI), lambda t, e, i: (e, 0, i)),
        ],
        out_specs=pl.BlockSpec((_BT, _HIDDEN), lambda t, e, i: (t, 0)),
        out_shape=jax.ShapeDtypeStruct((_TOKENS, _HIDDEN), jnp.float32),
        compiler_params=pltpu.CompilerParams(
            dimension_semantics=("parallel", "arbitrary", "arbitrary"),
        ),
    )(router_logits, x, w1, w3, w2)
